# baseline (speedup 1.0000x reference)
import numpy as np
import jax
import jax.numpy as jnp
from jax.experimental import pallas as pl
from jax.experimental.pallas import tpu as pltpu

Q, C, K = 256, 128, 65536
BK = 4096
NB = K // BK
NF = BK // 128
HB = NB // 2
HP, WP = 16, 16
IMG = 256
NBRS = 10
_BIG = 3.0e38


def _postproc_matrix() -> np.ndarray:
    i = np.arange(IMG, dtype=np.float64)
    x = (i + 0.5) * (HP / IMG) - 0.5
    j = np.arange(HP, dtype=np.float64)
    U = np.maximum(0.0, 1.0 - np.abs(x[:, None] - j[None, :]))
    U /= U.sum(axis=1, keepdims=True)
    sigma = 4.0
    radius = int(4.0 * sigma)
    t = np.arange(-radius, radius + 1, dtype=np.float64)
    g = np.exp(-0.5 * (t / sigma) ** 2)
    g /= g.sum()
    G = np.zeros((IMG, IMG))
    idx = np.arange(IMG)
    for d in range(-radius, radius + 1):
        src = idx + d
        m = (src >= 0) & (src < IMG)
        G[idx[m], src[m]] += g[d + radius]
    return (G @ U).astype(np.float32)


_M_NP = _postproc_matrix()
_MT_NP = np.ascontiguousarray(_M_NP.T)


def _lane_min_all(x, width=128):
    s = width // 2
    while s >= 1:
        x = jnp.minimum(x, pltpu.roll(x, s, 1))
        s //= 2
    return x


def _lane_sum_all(x, width=128):
    s = width // 2
    while s >= 1:
        x = x + pltpu.roll(x, s, 1)
        s //= 2
    return x


def _dot(a, b):
    return jax.lax.dot_general(
        a, b, (((1,), (1,)), ((), ())),
        preferred_element_type=jnp.float32,
        precision=jax.lax.Precision.HIGHEST)


def _body(q_ref, k_ref, m_ref, mt_ref, scores_ref, amap_ref,
          keys_s, kk_s, qk_s, md2_s, a_s, b_s, topk_s, minv_s, mini_s):
    i = pl.program_id(0)

    @pl.when(i < NB)
    def _mm():
        k = k_ref[...]
        ones = jnp.full((1, C), 1.0, jnp.float32)
        kkb = _dot(ones, k * k)
        qk_s[pl.ds(i % 2, 1), :, :] = _dot(q_ref[...], k)[None]
        keys_s[pl.ds(i * BK, BK), :] = k
        kk_s[pl.ds(i, 1), :] = kkb

    @pl.when(i > 0)
    def _fold():
        j = i - 1
        qk = qk_s[pl.ds(j % 2, 1), :, :][0]
        kkb = kk_s[pl.ds(j, 1), :]
        m128 = kkb[:, 0:128] - 2.0 * qk[:, 0:128]
        s128 = jnp.zeros((Q, 128), jnp.int32)
        for c in range(1, NF):
            sl = slice(c * 128, (c + 1) * 128)
            nxt = kkb[:, sl] - 2.0 * qk[:, sl]
            win = nxt < m128
            m128 = jnp.where(win, nxt, m128)
            s128 = jnp.where(win, c, s128)
        g128 = s128 + j * NF

        @pl.when(j == 0)
        def _():
            minv_s[...] = m128
            mini_s[...] = g128

        @pl.when(j > 0)
        def _():
            win2 = m128 < minv_s[...]
            minv_s[...] = jnp.where(win2, m128, minv_s[...])
            mini_s[...] = jnp.where(win2, g128, mini_s[...])

    @pl.when(i == NB)
    def _phase2():
        q = q_ref[...]
        qq = _lane_sum_all(q * q)
        gmin = _lane_min_all(minv_s[...])
        s2 = qq + gmin
        pd = jnp.sqrt(jnp.maximum(s2, 0.0))
        mx = jnp.max(s2)
        s_star = jnp.sqrt(jnp.maximum(mx, 0.0))
        rowio = jax.lax.broadcasted_iota(jnp.int32, (Q, 128), 0)
        lanio = jax.lax.broadcasted_iota(jnp.int32, (Q, 128), 1)
        qidx = jnp.min(jnp.where(s2 == mx, rowio, Q))
        qwin = (rowio == qidx) & (minv_s[...] == gmin)
        kidx = mini_s[...] * 128 + lanio
        midx = jnp.min(jnp.where(qwin, kidx, K))

        m_star = keys_s[pl.ds(midx, 1), :]
        mm = jnp.sum(m_star * m_star)

        def _scan_m(c, carry):
            kc = keys_s[pl.ds(c * BK, BK), :]
            md2_s[pl.ds(c, 1), :] = (kk_s[pl.ds(c, 1), :]
                                     - 2.0 * _dot(m_star, kc) + mm)
            return carry

        jax.lax.fori_loop(0, NB, _scan_m, 0)

        a_s[...] = jnp.minimum(md2_s[0:HB, :], md2_s[HB:NB, :])
        b_s[...] = jnp.maximum(md2_s[0:HB, :], md2_s[HB:NB, :])
        tlane = jax.lax.broadcasted_iota(jnp.int32, (1, 128), 1)

        def _sel(t, carry):
            cur = a_s[...]
            v = jnp.min(cur)
            hit = cur == v
            a_s[...] = jnp.where(hit, b_s[...], cur)
            b_s[...] = jnp.where(hit, _BIG, b_s[...])
            topk_s[...] = jnp.where(tlane == t, v, topk_s[...])
            return carry

        jax.lax.fori_loop(0, NBRS, _sel, 0)

        dtk = jnp.sqrt(jnp.maximum(topk_s[...], 0.0))
        d_last = jnp.sum(jnp.where(tlane == NBRS - 1, dtk, 0.0))
        e = jnp.exp(dtk - d_last)
        nb_mask = (tlane >= 1) & (tlane <= NBRS - 1)
        e_sum = jnp.sum(jnp.where(nb_mask, e, 0.0))
        e_1 = jnp.sum(jnp.where(tlane == 1, e, 0.0))
        w = 1.0 - e_1 / e_sum
        scores_ref[...] = jnp.full((1, 1), w * s_star, jnp.float32)

        qi = jax.lax.broadcasted_iota(jnp.int32, (Q, WP), 0)
        bi = jax.lax.broadcasted_iota(jnp.int32, (Q, WP), 1)
        F = jnp.where(qi % WP == bi, pd[:, 0:WP], 0.0)
        ai = jax.lax.broadcasted_iota(jnp.int32, (HP, Q), 0)
        qi2 = jax.lax.broadcasted_iota(jnp.int32, (HP, Q), 1)
        E = jnp.where(qi2 // WP == ai, 1.0, 0.0)
        P = jax.lax.dot_general(E, F, (((1,), (0,)), ((), ())),
                                preferred_element_type=jnp.float32,
                                precision=jax.lax.Precision.HIGHEST)
        A1 = jax.lax.dot_general(m_ref[...], P, (((1,), (0,)), ((), ())),
                                 preferred_element_type=jnp.float32,
                                 precision=jax.lax.Precision.HIGHEST)
        amap_ref[...] = jax.lax.dot_general(A1, mt_ref[...],
                                            (((1,), (0,)), ((), ())),
                                            preferred_element_type=jnp.float32,
                                            precision=jax.lax.Precision.HIGHEST)


def kernel(queries, keys):
    m_c = jnp.asarray(_M_NP)
    mt_c = jnp.asarray(_MT_NP)
    scores, amap = pl.pallas_call(
        _body,
        grid=(NB + 1,),
        in_specs=[
            pl.BlockSpec((Q, C), lambda i: (0, 0)),
            pl.BlockSpec((BK, C), lambda i: (jnp.minimum(i, NB - 1), 0)),
            pl.BlockSpec((IMG, HP), lambda i: (0, 0)),
            pl.BlockSpec((HP, IMG), lambda i: (0, 0)),
        ],
        out_specs=[
            pl.BlockSpec((1, 1), lambda i: (0, 0)),
            pl.BlockSpec((IMG, IMG), lambda i: (0, 0)),
        ],
        out_shape=[
            jax.ShapeDtypeStruct((1, 1), jnp.float32),
            jax.ShapeDtypeStruct((IMG, IMG), jnp.float32),
        ],
        scratch_shapes=[
            pltpu.VMEM((K, C), jnp.float32),
            pltpu.VMEM((NB, BK), jnp.float32),
            pltpu.VMEM((2, Q, BK), jnp.float32),
            pltpu.VMEM((NB, BK), jnp.float32),
            pltpu.VMEM((HB, BK), jnp.float32),
            pltpu.VMEM((HB, BK), jnp.float32),
            pltpu.VMEM((1, 128), jnp.float32),
            pltpu.VMEM((Q, 128), jnp.float32),
            pltpu.VMEM((Q, 128), jnp.int32),
        ],
        compiler_params=pltpu.CompilerParams(
            dimension_semantics=("arbitrary",),
            vmem_limit_bytes=60 * 1024 * 1024,
        ),
    )(queries, keys, m_c, mt_c)
    return scores.reshape(1), amap.reshape(1, IMG, IMG)


# speedup vs baseline: 1.3960x; 1.3960x over previous
import numpy as np
import jax
import jax.numpy as jnp
from jax.experimental import pallas as pl
from jax.experimental.pallas import tpu as pltpu

Q, C, K = 256, 128, 65536
BK = 4096
NB = K // BK
NF = BK // 128
HP, WP = 16, 16
IMG = 256
NBRS = 10
_BIG = 3.0e38


def _postproc_matrix() -> np.ndarray:
    i = np.arange(IMG, dtype=np.float64)
    x = (i + 0.5) * (HP / IMG) - 0.5
    j = np.arange(HP, dtype=np.float64)
    U = np.maximum(0.0, 1.0 - np.abs(x[:, None] - j[None, :]))
    U /= U.sum(axis=1, keepdims=True)
    sigma = 4.0
    radius = int(4.0 * sigma)
    t = np.arange(-radius, radius + 1, dtype=np.float64)
    g = np.exp(-0.5 * (t / sigma) ** 2)
    g /= g.sum()
    G = np.zeros((IMG, IMG))
    idx = np.arange(IMG)
    for d in range(-radius, radius + 1):
        src = idx + d
        m = (src >= 0) & (src < IMG)
        G[idx[m], src[m]] += g[d + radius]
    return (G @ U).astype(np.float32)


_M_NP = _postproc_matrix()
_MT_NP = np.ascontiguousarray(_M_NP.T)


def _lane_min_all(x, width=128):
    s = width // 2
    while s >= 1:
        x = jnp.minimum(x, pltpu.roll(x, s, 1))
        s //= 2
    return x


def _lane_sum_all(x, width=128):
    s = width // 2
    while s >= 1:
        x = x + pltpu.roll(x, s, 1)
        s //= 2
    return x


def _dot(a, b):
    return jax.lax.dot_general(
        a, b, (((1,), (1,)), ((), ())),
        preferred_element_type=jnp.float32,
        precision=jax.lax.Precision.HIGHEST)


def _body(q_ref, k_ref, m_ref, mt_ref, scores_ref, amap_ref,
          keys_s, kk_s, md2_s, a_s, b_s, topk_s, minv_s, mini_s):
    i = pl.program_id(0)

    @pl.when(i < NB)
    def _phase1():
        k = k_ref[...]
        ones = jnp.full((1, C), 1.0, jnp.float32)
        kkb = _dot(ones, k * k)
        qk = _dot(q_ref[...], k)
        m128 = kkb[:, 0:128] - 2.0 * qk[:, 0:128]
        s128 = jnp.zeros((Q, 128), jnp.int32)
        for c in range(1, NF):
            sl = slice(c * 128, (c + 1) * 128)
            nxt = kkb[:, sl] - 2.0 * qk[:, sl]
            win = nxt < m128
            m128 = jnp.where(win, nxt, m128)
            s128 = jnp.where(win, c, s128)
        g128 = s128 + i * NF
        keys_s[pl.ds(i * BK, BK), :] = k.astype(jnp.bfloat16)
        kk_s[pl.ds(i, 1), :] = kkb

        @pl.when(i == 0)
        def _():
            minv_s[...] = m128
            mini_s[...] = g128

        @pl.when(i > 0)
        def _():
            win2 = m128 < minv_s[...]
            minv_s[...] = jnp.where(win2, m128, minv_s[...])
            mini_s[...] = jnp.where(win2, g128, mini_s[...])

    @pl.when(i == NB)
    def _phase2():
        q = q_ref[...]
        qq = _lane_sum_all(q * q)
        gmin = _lane_min_all(minv_s[...])
        s2 = qq + gmin
        pd = jnp.sqrt(jnp.maximum(s2, 0.0))
        mx = jnp.max(s2)
        s_star = jnp.sqrt(jnp.maximum(mx, 0.0))
        rowio = jax.lax.broadcasted_iota(jnp.int32, (Q, 128), 0)
        lanio = jax.lax.broadcasted_iota(jnp.int32, (Q, 128), 1)
        qidx = jnp.min(jnp.where(s2 == mx, rowio, Q))
        qwin = (rowio == qidx) & (minv_s[...] == gmin)
        kidx = mini_s[...] * 128 + lanio
        midx = jnp.min(jnp.where(qwin, kidx, K))

        mbase = pl.multiple_of((midx // 16) * 16, 16)
        slab = keys_s[pl.ds(mbase, 16), :].astype(jnp.float32)
        mrem = midx % 16
        msf = jnp.zeros((1, C), jnp.float32)
        for r in range(16):
            msf = jnp.where(mrem == r, slab[r:r + 1, :], msf)
        m_star = msf.astype(jnp.bfloat16)
        mm = jnp.sum(msf * msf)

        def _scan_m(c, carry):
            kc = keys_s[pl.ds(c * BK, BK), :]
            km = jax.lax.dot_general(
                m_star, kc, (((1,), (1,)), ((), ())),
                preferred_element_type=jnp.float32)
            md2_s[pl.ds(c, 1), :] = kk_s[pl.ds(c, 1), :] - 2.0 * km + mm
            return carry

        jax.lax.fori_loop(0, NB, _scan_m, 0)

        a_s[...] = jnp.minimum(md2_s[0:NB // 2, :], md2_s[NB // 2:NB, :])
        b_s[...] = jnp.maximum(md2_s[0:NB // 2, :], md2_s[NB // 2:NB, :])
        tlane = jax.lax.broadcasted_iota(jnp.int32, (1, 128), 1)

        def _sel(t, carry):
            cur = a_s[...]
            v = jnp.min(cur)
            hit = cur == v
            a_s[...] = jnp.where(hit, b_s[...], cur)
            b_s[...] = jnp.where(hit, _BIG, b_s[...])
            topk_s[...] = jnp.where(tlane == t, v, topk_s[...])
            return carry

        jax.lax.fori_loop(0, NBRS, _sel, 0)

        dtk = jnp.sqrt(jnp.maximum(topk_s[...], 0.0))
        d_last = jnp.sum(jnp.where(tlane == NBRS - 1, dtk, 0.0))
        e = jnp.exp(dtk - d_last)
        nb_mask = (tlane >= 1) & (tlane <= NBRS - 1)
        e_sum = jnp.sum(jnp.where(nb_mask, e, 0.0))
        e_1 = jnp.sum(jnp.where(tlane == 1, e, 0.0))
        w = 1.0 - e_1 / e_sum
        scores_ref[...] = jnp.full((1, 1), w * s_star, jnp.float32)

        qi = jax.lax.broadcasted_iota(jnp.int32, (Q, WP), 0)
        bi = jax.lax.broadcasted_iota(jnp.int32, (Q, WP), 1)
        F = jnp.where(qi % WP == bi, pd[:, 0:WP], 0.0)
        ai = jax.lax.broadcasted_iota(jnp.int32, (HP, Q), 0)
        qi2 = jax.lax.broadcasted_iota(jnp.int32, (HP, Q), 1)
        E = jnp.where(qi2 // WP == ai, 1.0, 0.0)
        P = jax.lax.dot_general(E, F, (((1,), (0,)), ((), ())),
                                preferred_element_type=jnp.float32,
                                precision=jax.lax.Precision.HIGHEST)
        A1 = jax.lax.dot_general(m_ref[...], P, (((1,), (0,)), ((), ())),
                                 preferred_element_type=jnp.float32,
                                 precision=jax.lax.Precision.HIGHEST)
        amap_ref[...] = jax.lax.dot_general(A1, mt_ref[...],
                                            (((1,), (0,)), ((), ())),
                                            preferred_element_type=jnp.float32,
                                            precision=jax.lax.Precision.HIGHEST)


def kernel(queries, keys):
    m_c = jnp.asarray(_M_NP)
    mt_c = jnp.asarray(_MT_NP)
    scores, amap = pl.pallas_call(
        _body,
        grid=(NB + 1,),
        in_specs=[
            pl.BlockSpec((Q, C), lambda i: (0, 0)),
            pl.BlockSpec((BK, C), lambda i: (jnp.minimum(i, NB - 1), 0)),
            pl.BlockSpec((IMG, HP), lambda i: (0, 0)),
            pl.BlockSpec((HP, IMG), lambda i: (0, 0)),
        ],
        out_specs=[
            pl.BlockSpec((1, 1), lambda i: (0, 0)),
            pl.BlockSpec((IMG, IMG), lambda i: (0, 0)),
        ],
        out_shape=[
            jax.ShapeDtypeStruct((1, 1), jnp.float32),
            jax.ShapeDtypeStruct((IMG, IMG), jnp.float32),
        ],
        scratch_shapes=[
            pltpu.VMEM((K, C), jnp.bfloat16),
            pltpu.VMEM((NB, BK), jnp.float32),
            pltpu.VMEM((NB, BK), jnp.float32),
            pltpu.VMEM((NB // 2, BK), jnp.float32),
            pltpu.VMEM((NB // 2, BK), jnp.float32),
            pltpu.VMEM((1, 128), jnp.float32),
            pltpu.VMEM((Q, 128), jnp.float32),
            pltpu.VMEM((Q, 128), jnp.int32),
        ],
        compiler_params=pltpu.CompilerParams(
            dimension_semantics=("arbitrary",),
            vmem_limit_bytes=60 * 1024 * 1024,
        ),
    )(queries, keys, m_c, mt_c)
    return scores.reshape(1), amap.reshape(1, IMG, IMG)


# speedup vs baseline: 1.4322x; 1.0259x over previous
import numpy as np
import jax
import jax.numpy as jnp
from jax.experimental import pallas as pl
from jax.experimental.pallas import tpu as pltpu

Q, C, K = 256, 128, 65536
BK = 8192
NB = K // BK
NF = BK // 128
HP, WP = 16, 16
IMG = 256
NBRS = 10
_BIG = 3.0e38


def _postproc_matrix() -> np.ndarray:
    i = np.arange(IMG, dtype=np.float64)
    x = (i + 0.5) * (HP / IMG) - 0.5
    j = np.arange(HP, dtype=np.float64)
    U = np.maximum(0.0, 1.0 - np.abs(x[:, None] - j[None, :]))
    U /= U.sum(axis=1, keepdims=True)
    sigma = 4.0
    radius = int(4.0 * sigma)
    t = np.arange(-radius, radius + 1, dtype=np.float64)
    g = np.exp(-0.5 * (t / sigma) ** 2)
    g /= g.sum()
    G = np.zeros((IMG, IMG))
    idx = np.arange(IMG)
    for d in range(-radius, radius + 1):
        src = idx + d
        m = (src >= 0) & (src < IMG)
        G[idx[m], src[m]] += g[d + radius]
    return (G @ U).astype(np.float32)


_M_NP = _postproc_matrix()
_MT_NP = np.ascontiguousarray(_M_NP.T)


def _lane_min_all(x, width=128):
    s = width // 2
    while s >= 1:
        x = jnp.minimum(x, pltpu.roll(x, s, 1))
        s //= 2
    return x


def _lane_sum_all(x, width=128):
    s = width // 2
    while s >= 1:
        x = x + pltpu.roll(x, s, 1)
        s //= 2
    return x


def _dot(a, b):
    return jax.lax.dot_general(
        a, b, (((1,), (1,)), ((), ())),
        preferred_element_type=jnp.float32,
        precision=jax.lax.Precision.HIGHEST)


def _body(q_ref, k_ref, m_ref, mt_ref, scores_ref, amap_ref,
          keys_s, kk_s, md2_s, a_s, b_s, topk_s, minv_s, mini_s):
    i = pl.program_id(0)

    @pl.when(i < NB)
    def _phase1():
        k = k_ref[...]
        ones = jnp.full((1, C), 1.0, jnp.float32)
        kkb = _dot(ones, k * k)
        qk = _dot(q_ref[...], k)
        m128 = kkb[:, 0:128] - 2.0 * qk[:, 0:128]
        s128 = jnp.zeros((Q, 128), jnp.int32)
        for c in range(1, NF):
            sl = slice(c * 128, (c + 1) * 128)
            nxt = kkb[:, sl] - 2.0 * qk[:, sl]
            win = nxt < m128
            m128 = jnp.where(win, nxt, m128)
            s128 = jnp.where(win, c, s128)
        g128 = s128 + i * NF
        keys_s[pl.ds(i * BK, BK), :] = k.astype(jnp.bfloat16)
        kk_s[pl.ds(i, 1), :] = kkb

        @pl.when(i == 0)
        def _():
            minv_s[...] = m128
            mini_s[...] = g128

        @pl.when(i > 0)
        def _():
            win2 = m128 < minv_s[...]
            minv_s[...] = jnp.where(win2, m128, minv_s[...])
            mini_s[...] = jnp.where(win2, g128, mini_s[...])

    @pl.when(i == NB)
    def _phase2():
        q = q_ref[...]
        qq = _lane_sum_all(q * q)
        gmin = _lane_min_all(minv_s[...])
        s2 = qq + gmin
        pd = jnp.sqrt(jnp.maximum(s2, 0.0))
        mx = jnp.max(s2)
        s_star = jnp.sqrt(jnp.maximum(mx, 0.0))
        rowio = jax.lax.broadcasted_iota(jnp.int32, (Q, 128), 0)
        lanio = jax.lax.broadcasted_iota(jnp.int32, (Q, 128), 1)
        qidx = jnp.min(jnp.where(s2 == mx, rowio, Q))
        qwin = (rowio == qidx) & (minv_s[...] == gmin)
        kidx = mini_s[...] * 128 + lanio
        midx = jnp.min(jnp.where(qwin, kidx, K))

        mbase = pl.multiple_of((midx // 16) * 16, 16)
        slab = keys_s[pl.ds(mbase, 16), :].astype(jnp.float32)
        mrem = midx % 16
        msf = jnp.zeros((1, C), jnp.float32)
        for r in range(16):
            msf = jnp.where(mrem == r, slab[r:r + 1, :], msf)
        m_star = msf.astype(jnp.bfloat16)
        mm = jnp.sum(msf * msf)

        def _scan_m(c, carry):
            kc = keys_s[pl.ds(c * BK, BK), :]
            km = jax.lax.dot_general(
                m_star, kc, (((1,), (1,)), ((), ())),
                preferred_element_type=jnp.float32)
            md2_s[pl.ds(c, 1), :] = kk_s[pl.ds(c, 1), :] - 2.0 * km + mm
            return carry

        jax.lax.fori_loop(0, NB, _scan_m, 0)

        a_s[...] = jnp.minimum(md2_s[0:NB // 2, :], md2_s[NB // 2:NB, :])
        b_s[...] = jnp.maximum(md2_s[0:NB // 2, :], md2_s[NB // 2:NB, :])
        tlane = jax.lax.broadcasted_iota(jnp.int32, (1, 128), 1)

        def _sel(t, carry):
            cur = a_s[...]
            v = jnp.min(cur)
            hit = cur == v
            a_s[...] = jnp.where(hit, b_s[...], cur)
            b_s[...] = jnp.where(hit, _BIG, b_s[...])
            topk_s[...] = jnp.where(tlane == t, v, topk_s[...])
            return carry

        jax.lax.fori_loop(0, NBRS, _sel, 0)

        dtk = jnp.sqrt(jnp.maximum(topk_s[...], 0.0))
        d_last = jnp.sum(jnp.where(tlane == NBRS - 1, dtk, 0.0))
        e = jnp.exp(dtk - d_last)
        nb_mask = (tlane >= 1) & (tlane <= NBRS - 1)
        e_sum = jnp.sum(jnp.where(nb_mask, e, 0.0))
        e_1 = jnp.sum(jnp.where(tlane == 1, e, 0.0))
        w = 1.0 - e_1 / e_sum
        scores_ref[...] = jnp.full((1, 1), w * s_star, jnp.float32)

        qi = jax.lax.broadcasted_iota(jnp.int32, (Q, WP), 0)
        bi = jax.lax.broadcasted_iota(jnp.int32, (Q, WP), 1)
        F = jnp.where(qi % WP == bi, pd[:, 0:WP], 0.0)
        ai = jax.lax.broadcasted_iota(jnp.int32, (HP, Q), 0)
        qi2 = jax.lax.broadcasted_iota(jnp.int32, (HP, Q), 1)
        E = jnp.where(qi2 // WP == ai, 1.0, 0.0)
        P = jax.lax.dot_general(E, F, (((1,), (0,)), ((), ())),
                                preferred_element_type=jnp.float32,
                                precision=jax.lax.Precision.HIGHEST)
        A1 = jax.lax.dot_general(m_ref[...], P, (((1,), (0,)), ((), ())),
                                 preferred_element_type=jnp.float32,
                                 precision=jax.lax.Precision.HIGHEST)
        amap_ref[...] = jax.lax.dot_general(A1, mt_ref[...],
                                            (((1,), (0,)), ((), ())),
                                            preferred_element_type=jnp.float32,
                                            precision=jax.lax.Precision.HIGHEST)


def kernel(queries, keys):
    m_c = jnp.asarray(_M_NP)
    mt_c = jnp.asarray(_MT_NP)
    scores, amap = pl.pallas_call(
        _body,
        grid=(NB + 1,),
        in_specs=[
            pl.BlockSpec((Q, C), lambda i: (0, 0)),
            pl.BlockSpec((BK, C), lambda i: (jnp.minimum(i, NB - 1), 0)),
            pl.BlockSpec((IMG, HP), lambda i: (0, 0)),
            pl.BlockSpec((HP, IMG), lambda i: (0, 0)),
        ],
        out_specs=[
            pl.BlockSpec((1, 1), lambda i: (0, 0)),
            pl.BlockSpec((IMG, IMG), lambda i: (0, 0)),
        ],
        out_shape=[
            jax.ShapeDtypeStruct((1, 1), jnp.float32),
            jax.ShapeDtypeStruct((IMG, IMG), jnp.float32),
        ],
        scratch_shapes=[
            pltpu.VMEM((K, C), jnp.bfloat16),
            pltpu.VMEM((NB, BK), jnp.float32),
            pltpu.VMEM((NB, BK), jnp.float32),
            pltpu.VMEM((NB // 2, BK), jnp.float32),
            pltpu.VMEM((NB // 2, BK), jnp.float32),
            pltpu.VMEM((1, 128), jnp.float32),
            pltpu.VMEM((Q, 128), jnp.float32),
            pltpu.VMEM((Q, 128), jnp.int32),
        ],
        compiler_params=pltpu.CompilerParams(
            dimension_semantics=("arbitrary",),
            vmem_limit_bytes=60 * 1024 * 1024,
        ),
    )(queries, keys, m_c, mt_c)
    return scores.reshape(1), amap.reshape(1, IMG, IMG)
